# Initial kernel scaffold; baseline (speedup 1.0000x reference)
import jax
import jax.numpy as jnp
from jax.experimental import pallas as pl


def kernel(x, edge_index, edge_weight):
    raise NotImplementedError("write your pallas kernel here")


# speedup vs baseline: 6.3921x; 6.3921x over previous
import functools

import jax
import jax.numpy as jnp
from jax import lax
from jax.experimental import pallas as pl
from jax.experimental.pallas import tpu as pltpu
from jax.experimental.pallas import tpu_sc as plsc

NC = 2
NS = 16
LANES = 16
NW = NC * NS
K = 80


def _sc_partials(x, src4, dst4, w4, zeros):
    N, D = x.shape
    _, SUP, SCH, _ = src4.shape
    rows_per_tile = N // NS
    fgroups = D // LANES

    mesh = plsc.VectorSubcoreMesh(core_axis_name="c", subcore_axis_name="s")

    @functools.partial(
        pl.kernel,
        out_type=jax.ShapeDtypeStruct((NC, NS, N // NS, D), jnp.float32),
        mesh=mesh,
        scratch_types=[
            pltpu.VMEM((SCH, K), jnp.int32),
            pltpu.VMEM((SCH, K), jnp.int32),
            pltpu.VMEM((SCH, K), jnp.float32),
            pltpu.VMEM((K, D), jnp.float32),
            pltpu.VMEM_SHARED((N, D), jnp.float32),
            pltpu.SemaphoreType.DMA,
        ],
    )
    def k(x_hbm, src_hbm, dst_hbm, w_hbm, z_hbm, part_hbm,
          src_v, dst_v, w_v, rows_v, acc_sh, sem):
        c = lax.axis_index("c")
        s = lax.axis_index("s")
        wid = c * NS + s
        r0 = s * rows_per_tile

        pltpu.sync_copy(z_hbm.at[s], acc_sh.at[pl.ds(r0, rows_per_tile)])
        plsc.subcore_barrier()

        def superchunk(u, carry0):
            pltpu.sync_copy(src_hbm.at[wid, u], src_v)
            pltpu.sync_copy(dst_hbm.at[wid, u], dst_v)
            pltpu.sync_copy(w_hbm.at[wid, u], w_v)

            def chunk(t, carry):
                pltpu.async_copy(x_hbm.at[src_v.at[t]], rows_v, sem).wait()

                def group(g, carry2):
                    wg = w_v[t, pl.ds(g * LANES, LANES)]
                    for i in range(LANES):
                        ws = wg.at[jnp.full((LANES,), i, jnp.int32)].get(
                            mode="promise_in_bounds")
                        e = g * LANES + i
                        for f in range(fgroups):
                            sl = pl.ds(f * LANES, LANES)
                            rows_v[e, sl] = rows_v[e, sl] * ws
                    return carry2

                lax.fori_loop(0, K // LANES, group, 0)
                pltpu.sync_copy(rows_v, acc_sh.at[dst_v.at[t]], add=True)
                return carry

            lax.fori_loop(0, SCH, chunk, 0)
            return carry0

        lax.fori_loop(0, SUP, superchunk, 0)
        plsc.subcore_barrier()
        pltpu.sync_copy(acc_sh.at[pl.ds(r0, rows_per_tile)],
                        part_hbm.at[c, s])

    return k(x, src4, dst4, w4, zeros)


def _finish_tc(parts):
    ncp, N, D = parts.shape
    blk = 1000

    def body(p_ref, o_ref):
        h = p_ref[0] + p_ref[1]
        n2 = jnp.sum(h * h, axis=1, keepdims=True)
        o_ref[...] = h * lax.rsqrt(jnp.maximum(n2, 1e-24))

    return pl.pallas_call(
        body,
        grid=(N // blk,),
        in_specs=[pl.BlockSpec((ncp, blk, D), lambda i: (0, i, 0))],
        out_specs=pl.BlockSpec((blk, D), lambda i: (i, 0)),
        out_shape=jax.ShapeDtypeStruct((N, D), jnp.float32),
    )(parts)


def kernel(x, edge_index, edge_weight):
    N, D = x.shape
    E = edge_index.shape[1]
    epw = E // NW
    ch = epw // K
    sup, sch = 5, ch // 5
    src4 = edge_index[0].reshape(NW, sup, sch, K)
    dst4 = edge_index[1].reshape(NW, sup, sch, K)
    w4 = edge_weight.reshape(NW, sup, sch, K)
    zeros = jnp.zeros((NS, N // NS, D), jnp.float32)
    parts = _sc_partials(x, src4, dst4, w4, zeros)
    return _finish_tc(parts.reshape(NC, N, D))


# speedup vs baseline: 9.1573x; 1.4326x over previous
import functools

import jax
import jax.numpy as jnp
from jax import lax
from jax.experimental import pallas as pl
from jax.experimental.pallas import tpu as pltpu
from jax.experimental.pallas import tpu_sc as plsc

NC = 2
NS = 16
LANES = 16
NW = NC * NS
K = 80
SCH = 16
SUP = 8


def _sc_partials(x, src4, dst4, w4, zeros):
    N, D = x.shape
    rows_per_tile = N // NS
    fgroups = D // LANES

    mesh = plsc.VectorSubcoreMesh(core_axis_name="c", subcore_axis_name="s")

    @functools.partial(
        pl.kernel,
        out_type=jax.ShapeDtypeStruct((NC, NS, N // NS, D), jnp.float32),
        mesh=mesh,
        scratch_types=[
            pltpu.VMEM((SCH, K), jnp.int32),
            pltpu.VMEM((SCH, K), jnp.int32),
            pltpu.VMEM((SCH, K), jnp.float32),
            pltpu.VMEM((2, K, D), jnp.float32),
            pltpu.VMEM_SHARED((N, D), jnp.float32),
            pltpu.SemaphoreType.DMA,
            pltpu.SemaphoreType.DMA,
        ],
    )
    def k(x_hbm, src_hbm, dst_hbm, w_hbm, z_hbm, part_hbm,
          src_v, dst_v, w_v, rows_v, acc_sh, sem0, sem1):
        c = lax.axis_index("c")
        s = lax.axis_index("s")
        wid = c * NS + s
        r0 = s * rows_per_tile
        sems = (sem0, sem1)

        pltpu.sync_copy(z_hbm.at[s], acc_sh.at[pl.ds(r0, rows_per_tile)])
        plsc.subcore_barrier()

        def start_gather(t, b):
            pltpu.async_copy(x_hbm.at[src_v.at[t]], rows_v.at[b], sems[b])

        def finish_chunk(t, b):
            pltpu.make_async_copy(x_hbm.at[src_v.at[t]], rows_v.at[b],
                                  sems[b]).wait()
            buf = rows_v.at[b]

            def group(g, carry2):
                wg = w_v[t, pl.ds(g * LANES, LANES)]
                for i in range(LANES):
                    ws = wg.at[jnp.full((LANES,), i, jnp.int32)].get(
                        mode="promise_in_bounds")
                    e = g * LANES + i
                    for f in range(fgroups):
                        sl = pl.ds(f * LANES, LANES)
                        buf[e, sl] = buf[e, sl] * ws
                return carry2

            lax.fori_loop(0, K // LANES, group, 0)
            pltpu.sync_copy(buf, acc_sh.at[dst_v.at[t]], add=True)

        def superchunk(u, carry0):
            pltpu.sync_copy(src_hbm.at[wid, u], src_v)
            pltpu.sync_copy(dst_hbm.at[wid, u], dst_v)
            pltpu.sync_copy(w_hbm.at[wid, u], w_v)
            start_gather(0, 0)

            def pair(p, carry):
                t0 = 2 * p
                start_gather(t0 + 1, 1)
                finish_chunk(t0, 0)

                @pl.when(p != SCH // 2 - 1)
                def _():
                    start_gather(t0 + 2, 0)

                finish_chunk(t0 + 1, 1)
                return carry

            lax.fori_loop(0, SCH // 2, pair, 0)
            return carry0

        lax.fori_loop(0, SUP, superchunk, 0)
        plsc.subcore_barrier()
        pltpu.sync_copy(acc_sh.at[pl.ds(r0, rows_per_tile)],
                        part_hbm.at[c, s])

    return k(x, src4, dst4, w4, zeros)


def _finish_tc(parts):
    ncp, N, D = parts.shape
    blk = 1000

    def body(p_ref, o_ref):
        h = p_ref[0] + p_ref[1]
        n2 = jnp.sum(h * h, axis=1, keepdims=True)
        o_ref[...] = h * lax.rsqrt(jnp.maximum(n2, 1e-24))

    return pl.pallas_call(
        body,
        grid=(N // blk,),
        in_specs=[pl.BlockSpec((ncp, blk, D), lambda i: (0, i, 0))],
        out_specs=pl.BlockSpec((blk, D), lambda i: (i, 0)),
        out_shape=jax.ShapeDtypeStruct((N, D), jnp.float32),
    )(parts)


def kernel(x, edge_index, edge_weight):
    N, D = x.shape
    E = edge_index.shape[1]
    e_pad = NW * SUP * SCH * K
    pad = e_pad - E
    pad_idx = (jnp.arange(pad, dtype=jnp.int32) * 37) % N
    src_p = jnp.concatenate([edge_index[0], pad_idx]).reshape(NW, SUP, SCH, K)
    dst_p = jnp.concatenate([edge_index[1], pad_idx]).reshape(NW, SUP, SCH, K)
    w_p = jnp.concatenate(
        [edge_weight, jnp.zeros((pad,), jnp.float32)]).reshape(NW, SUP, SCH, K)
    zeros = jnp.zeros((NS, N // NS, D), jnp.float32)
    parts = _sc_partials(x, src_p, dst_p, w_p, zeros)
    return _finish_tc(parts.reshape(NC, N, D))


# speedup vs baseline: 9.5138x; 1.0389x over previous
import functools

import jax
import jax.numpy as jnp
from jax import lax
from jax.experimental import pallas as pl
from jax.experimental.pallas import tpu as pltpu
from jax.experimental.pallas import tpu_sc as plsc

NC = 2
NS = 16
LANES = 16
NW = NC * NS
K = 64
SCH = 18
SUP = 9
NBUF = 3


def _sc_partials(x, src4, dst4, w4, zeros):
    N, D = x.shape
    rows_per_tile = N // NS
    fgroups = D // LANES

    mesh = plsc.VectorSubcoreMesh(core_axis_name="c", subcore_axis_name="s")

    @functools.partial(
        pl.kernel,
        out_type=jax.ShapeDtypeStruct((NC, NS, N // NS, D), jnp.float32),
        mesh=mesh,
        scratch_types=[
            pltpu.VMEM((SCH, K), jnp.int32),
            pltpu.VMEM((SCH, K), jnp.int32),
            pltpu.VMEM((SCH, K), jnp.float32),
            pltpu.VMEM((NBUF, K, D), jnp.float32),
            pltpu.VMEM_SHARED((N, D), jnp.float32),
            [pltpu.SemaphoreType.DMA] * NBUF,
            [pltpu.SemaphoreType.DMA] * NBUF,
        ],
    )
    def k(x_hbm, src_hbm, dst_hbm, w_hbm, z_hbm, part_hbm,
          src_v, dst_v, w_v, rows_v, acc_sh, gsems, ssems):
        c = lax.axis_index("c")
        s = lax.axis_index("s")
        wid = c * NS + s
        r0 = s * rows_per_tile

        pltpu.sync_copy(z_hbm.at[s], acc_sh.at[pl.ds(r0, rows_per_tile)])
        plsc.subcore_barrier()

        def start_gather(t, b):
            pltpu.async_copy(x_hbm.at[src_v.at[t]], rows_v.at[b], gsems[b])

        def wait_gather(t, b):
            pltpu.make_async_copy(x_hbm.at[src_v.at[t]], rows_v.at[b],
                                  gsems[b]).wait()

        def start_scatter(t, b):
            pltpu.async_copy(rows_v.at[b], acc_sh.at[dst_v.at[t]], ssems[b],
                             add=True)

        def wait_scatter(b):
            pltpu.make_async_copy(rows_v.at[b], acc_sh.at[dst_v.at[0]],
                                  ssems[b]).wait()

        def compute(t, b):
            buf = rows_v.at[b]

            def group(g, carry2):
                wg = w_v[t, pl.ds(g * LANES, LANES)]
                for i in range(LANES):
                    ws = wg.at[jnp.full((LANES,), i, jnp.int32)].get(
                        mode="promise_in_bounds")
                    e = g * LANES + i
                    for f in range(fgroups):
                        sl = pl.ds(f * LANES, LANES)
                        buf[e, sl] = buf[e, sl] * ws
                return carry2

            lax.fori_loop(0, K // LANES, group, 0)

        def superchunk(u, carry0):
            pltpu.sync_copy(src_hbm.at[wid, u], src_v)
            pltpu.sync_copy(dst_hbm.at[wid, u], dst_v)
            pltpu.sync_copy(w_hbm.at[wid, u], w_v)
            start_gather(0, 0)
            start_gather(1, 1)

            def triple(t3, carry):
                for j in range(NBUF):
                    t = NBUF * t3 + j
                    b2 = (j + 2) % NBUF

                    @pl.when((t >= 1) & (t <= SCH - 3))
                    def _():
                        wait_scatter(b2)

                    @pl.when(t <= SCH - 3)
                    def _():
                        start_gather(t + 2, b2)

                    wait_gather(t, j)
                    compute(t, j)
                    start_scatter(t, j)
                return carry

            lax.fori_loop(0, SCH // NBUF, triple, 0)
            for b in range(NBUF):
                wait_scatter(b)
            return carry0

        lax.fori_loop(0, SUP, superchunk, 0)
        plsc.subcore_barrier()
        pltpu.sync_copy(acc_sh.at[pl.ds(r0, rows_per_tile)],
                        part_hbm.at[c, s])

    return k(x, src4, dst4, w4, zeros)


def _finish_tc(parts):
    ncp, N, D = parts.shape
    blk = 1000

    def body(p_ref, o_ref):
        h = p_ref[0] + p_ref[1]
        n2 = jnp.sum(h * h, axis=1, keepdims=True)
        o_ref[...] = h * lax.rsqrt(jnp.maximum(n2, 1e-24))

    return pl.pallas_call(
        body,
        grid=(N // blk,),
        in_specs=[pl.BlockSpec((ncp, blk, D), lambda i: (0, i, 0))],
        out_specs=pl.BlockSpec((blk, D), lambda i: (i, 0)),
        out_shape=jax.ShapeDtypeStruct((N, D), jnp.float32),
    )(parts)


def kernel(x, edge_index, edge_weight):
    N, D = x.shape
    E = edge_index.shape[1]
    e_pad = NW * SUP * SCH * K
    pad = e_pad - E
    pad_idx = (jnp.arange(pad, dtype=jnp.int32) * 37) % N
    src_p = jnp.concatenate([edge_index[0], pad_idx]).reshape(NW, SUP, SCH, K)
    dst_p = jnp.concatenate([edge_index[1], pad_idx]).reshape(NW, SUP, SCH, K)
    w_p = jnp.concatenate(
        [edge_weight, jnp.zeros((pad,), jnp.float32)]).reshape(NW, SUP, SCH, K)
    zeros = jnp.zeros((NS, N // NS, D), jnp.float32)
    parts = _sc_partials(x, src_p, dst_p, w_p, zeros)
    return _finish_tc(parts.reshape(NC, N, D))


# speedup vs baseline: 9.9679x; 1.0477x over previous
import functools

import jax
import jax.numpy as jnp
from jax import lax
from jax.experimental import pallas as pl
from jax.experimental.pallas import tpu as pltpu
from jax.experimental.pallas import tpu_sc as plsc

NC = 2
NS = 16
LANES = 16
NW = NC * NS
K = 80
SCH = 25
SUP = 5


def _sc_partials(x, src4, dst4, w4):
    N, D = x.shape
    rows_per_tile = N // NS
    fgroups = D // LANES

    mesh = plsc.VectorSubcoreMesh(core_axis_name="c", subcore_axis_name="s")

    @functools.partial(
        pl.kernel,
        out_type=jax.ShapeDtypeStruct((NC, NS, N // NS, D), jnp.float32),
        mesh=mesh,
        scratch_types=[
            pltpu.VMEM((SCH, K), jnp.int32),
            pltpu.VMEM((SCH, K), jnp.int32),
            pltpu.VMEM((SCH, K), jnp.float32),
            pltpu.VMEM((2, K, D), jnp.float32),
            pltpu.VMEM_SHARED((N, D), jnp.float32),
            pltpu.SemaphoreType.DMA,
            pltpu.SemaphoreType.DMA,
        ],
    )
    def k(x_hbm, src_hbm, dst_hbm, w_hbm, part_hbm,
          src_v, dst_v, w_v, rows_v, acc_sh, sem0, sem1):
        c = lax.axis_index("c")
        s = lax.axis_index("s")
        wid = c * NS + s
        r0 = s * rows_per_tile
        sems = (sem0, sem1)

        zbuf = rows_v.at[0]

        def zrow(r, carry):
            for f in range(fgroups):
                zbuf[r, pl.ds(f * LANES, LANES)] = jnp.zeros(
                    (LANES,), jnp.float32)
            return carry

        lax.fori_loop(0, K, zrow, 0)
        full, rem = divmod(rows_per_tile, K)
        for j in range(full):
            pltpu.sync_copy(zbuf, acc_sh.at[pl.ds(r0 + j * K, K)])
        if rem:
            pltpu.sync_copy(zbuf.at[pl.ds(0, rem)],
                            acc_sh.at[pl.ds(r0 + full * K, rem)])
        plsc.subcore_barrier()

        def start_gather(t, b):
            pltpu.async_copy(x_hbm.at[src_v.at[t]], rows_v.at[b], sems[b])

        def finish_chunk(t, b):
            pltpu.make_async_copy(x_hbm.at[src_v.at[t]], rows_v.at[b],
                                  sems[b]).wait()
            buf = rows_v.at[b]

            def group(g, carry2):
                wg = w_v[t, pl.ds(g * LANES, LANES)]
                for i in range(LANES):
                    ws = wg.at[jnp.full((LANES,), i, jnp.int32)].get(
                        mode="promise_in_bounds")
                    e = g * LANES + i
                    for f in range(fgroups):
                        sl = pl.ds(f * LANES, LANES)
                        buf[e, sl] = buf[e, sl] * ws
                return carry2

            lax.fori_loop(0, K // LANES, group, 0)
            pltpu.sync_copy(buf, acc_sh.at[dst_v.at[t]], add=True)

        def superchunk(u, carry0):
            pltpu.sync_copy(src_hbm.at[wid, u], src_v)
            pltpu.sync_copy(dst_hbm.at[wid, u], dst_v)
            pltpu.sync_copy(w_hbm.at[wid, u], w_v)
            start_gather(0, 0)

            def pair(p, carry):
                t0 = 2 * p
                start_gather(t0 + 1, 1)
                finish_chunk(t0, 0)

                @pl.when(t0 + 2 <= SCH - 1)
                def _():
                    start_gather(t0 + 2, 0)

                finish_chunk(t0 + 1, 1)
                return carry

            lax.fori_loop(0, SCH // 2, pair, 0)
            finish_chunk(SCH - 1, 0)
            return carry0

        lax.fori_loop(0, SUP, superchunk, 0)
        plsc.subcore_barrier()
        pltpu.sync_copy(acc_sh.at[pl.ds(r0, rows_per_tile)],
                        part_hbm.at[c, s])

    return k(x, src4, dst4, w4)


def _finish_tc(parts):
    ncp, N, D = parts.shape
    blk = 1000

    def body(p_ref, o_ref):
        h = p_ref[0] + p_ref[1]
        n2 = jnp.sum(h * h, axis=1, keepdims=True)
        o_ref[...] = h * lax.rsqrt(jnp.maximum(n2, 1e-24))

    return pl.pallas_call(
        body,
        grid=(N // blk,),
        in_specs=[pl.BlockSpec((ncp, blk, D), lambda i: (0, i, 0))],
        out_specs=pl.BlockSpec((blk, D), lambda i: (i, 0)),
        out_shape=jax.ShapeDtypeStruct((N, D), jnp.float32),
    )(parts)


def kernel(x, edge_index, edge_weight):
    N, D = x.shape
    src4 = edge_index[0].reshape(NW, SUP, SCH, K)
    dst4 = edge_index[1].reshape(NW, SUP, SCH, K)
    w4 = edge_weight.reshape(NW, SUP, SCH, K)
    parts = _sc_partials(x, src4, dst4, w4)
    return _finish_tc(parts.reshape(NC, N, D))


# speedup vs baseline: 10.9687x; 1.1004x over previous
import functools

import jax
import jax.numpy as jnp
from jax import lax
from jax.experimental import pallas as pl
from jax.experimental.pallas import tpu as pltpu
from jax.experimental.pallas import tpu_sc as plsc

NC = 2
NS = 16
LANES = 16
NW = NC * NS
K = 80
SCH = 25
SUP = 5


def _sc_partials(x, ei5, w4):
    N, D = x.shape
    rows_per_tile = N // NS
    fgroups = D // LANES
    wb = rows_per_tile // 8 * 8
    wrem = N - NS * wb

    mesh = plsc.VectorSubcoreMesh(core_axis_name="c", subcore_axis_name="s")

    @functools.partial(
        pl.kernel,
        out_type=jax.ShapeDtypeStruct((NC, N, D), jnp.float32),
        mesh=mesh,
        scratch_types=[
            pltpu.VMEM((SCH, K), jnp.int32),
            pltpu.VMEM((SCH, K), jnp.int32),
            pltpu.VMEM((SCH, K), jnp.float32),
            pltpu.VMEM((2, K, D), jnp.float32),
            pltpu.VMEM_SHARED((N, D), jnp.float32),
            pltpu.SemaphoreType.DMA,
            pltpu.SemaphoreType.DMA,
        ],
    )
    def k(x_hbm, ei_hbm, w_hbm, part_hbm,
          src_v, dst_v, w_v, rows_v, acc_sh, sem0, sem1):
        c = lax.axis_index("c")
        s = lax.axis_index("s")
        wid = c * NS + s
        r0 = s * rows_per_tile
        sems = (sem0, sem1)

        zbuf = rows_v.at[0]

        def zrow(r, carry):
            for f in range(fgroups):
                zbuf[r, pl.ds(f * LANES, LANES)] = jnp.zeros(
                    (LANES,), jnp.float32)
            return carry

        lax.fori_loop(0, K, zrow, 0)
        full, rem = divmod(rows_per_tile, K)
        for j in range(full):
            pltpu.sync_copy(zbuf, acc_sh.at[pl.ds(r0 + j * K, K)])
        if rem:
            pltpu.sync_copy(zbuf.at[pl.ds(0, rem)],
                            acc_sh.at[pl.ds(r0 + full * K, rem)])
        plsc.subcore_barrier()

        def start_gather(t, b):
            pltpu.async_copy(x_hbm.at[src_v.at[t]], rows_v.at[b], sems[b])

        def finish_chunk(t, b):
            pltpu.make_async_copy(x_hbm.at[src_v.at[t]], rows_v.at[b],
                                  sems[b]).wait()
            buf = rows_v.at[b]

            def group(g, carry2):
                wg = w_v[t, pl.ds(g * LANES, LANES)]
                for i in range(LANES):
                    ws = wg.at[jnp.full((LANES,), i, jnp.int32)].get(
                        mode="promise_in_bounds")
                    e = g * LANES + i
                    for f in range(fgroups):
                        sl = pl.ds(f * LANES, LANES)
                        buf[e, sl] = buf[e, sl] * ws
                return carry2

            lax.fori_loop(0, K // LANES, group, 0)
            pltpu.sync_copy(buf, acc_sh.at[dst_v.at[t]], add=True)

        def superchunk(u, carry0):
            pltpu.sync_copy(ei_hbm.at[0, wid, u], src_v)
            pltpu.sync_copy(ei_hbm.at[1, wid, u], dst_v)
            pltpu.sync_copy(w_hbm.at[wid, u], w_v)
            start_gather(0, 0)

            def pair(p, carry):
                t0 = 2 * p
                start_gather(t0 + 1, 1)
                finish_chunk(t0, 0)

                @pl.when(t0 + 2 <= SCH - 1)
                def _():
                    start_gather(t0 + 2, 0)

                finish_chunk(t0 + 1, 1)
                return carry

            lax.fori_loop(0, SCH // 2, pair, 0)
            finish_chunk(SCH - 1, 0)
            return carry0

        lax.fori_loop(0, SUP, superchunk, 0)
        plsc.subcore_barrier()
        base = pl.multiple_of(s * wb, 8)
        pltpu.sync_copy(acc_sh.at[pl.ds(base, wb)],
                        part_hbm.at[c, pl.ds(base, wb)])
        if wrem:
            @pl.when(s == NS - 1)
            def _():
                pltpu.sync_copy(acc_sh.at[pl.ds(NS * wb, wrem)],
                                part_hbm.at[c, pl.ds(NS * wb, wrem)])

    return k(x, ei5, w4)


def _finish_tc(parts):
    ncp, N, D = parts.shape
    blk = 1000

    def body(p_ref, o_ref):
        h = p_ref[0] + p_ref[1]
        n2 = jnp.sum(h * h, axis=1, keepdims=True)
        o_ref[...] = h * lax.rsqrt(jnp.maximum(n2, 1e-24))

    return pl.pallas_call(
        body,
        grid=(N // blk,),
        in_specs=[pl.BlockSpec((ncp, blk, D), lambda i: (0, i, 0))],
        out_specs=pl.BlockSpec((blk, D), lambda i: (i, 0)),
        out_shape=jax.ShapeDtypeStruct((N, D), jnp.float32),
    )(parts)


def kernel(x, edge_index, edge_weight):
    ei5 = edge_index.reshape(2, NW, SUP, SCH, K)
    w4 = edge_weight.reshape(NW, SUP, SCH, K)
    parts = _sc_partials(x, ei5, w4)
    return _finish_tc(parts)
